# baseline (speedup 1.0000x reference)
import functools

import jax
import jax.numpy as jnp
from jax import lax
from jax.experimental import pallas as pl
from jax.experimental.pallas import tpu as pltpu
from jax.experimental.pallas import tpu_sc as plsc

VOCAB = 1000000
EMB = 32
B = 16384
L = 20
N = B * L

NUM_CORES = 2
NUM_SUBCORES = 16
NW = NUM_CORES * NUM_SUBCORES
B_PER_W = B // NW
ROWS_PER_W = B_PER_W * L
CHUNK_B = 32
CHUNK = CHUNK_B * L
N_CHUNKS = B_PER_W // CHUNK_B
NGB = CHUNK_B // 16


def _make_gather():
    mesh = plsc.VectorSubcoreMesh(core_axis_name="c", subcore_axis_name="s")

    @functools.partial(
        pl.kernel,
        mesh=mesh,
        out_type=jax.ShapeDtypeStruct((L, EMB, B), jnp.float32),
        scratch_types=[
            pltpu.VMEM((ROWS_PER_W,), jnp.int32),
            pltpu.VMEM((2, CHUNK, EMB), jnp.float32),
            pltpu.VMEM((2, L, EMB, CHUNK_B), jnp.float32),
            pltpu.SemaphoreType.DMA((2,)),
            pltpu.SemaphoreType.DMA((2,)),
        ],
        compiler_params=pltpu.CompilerParams(
            use_tc_tiling_on_sc=False, needs_layout_passes=False
        ),
    )
    def gather_kernel(idx_hbm, table_hbm, out_hbm, idx_v, rows_v, t_v, gsem, wsem):
        wid = lax.axis_index("s") * NUM_CORES + lax.axis_index("c")
        base = wid * ROWS_PER_W
        b_base = wid * B_PER_W
        pltpu.sync_copy(idx_hbm.at[pl.ds(base, ROWS_PER_W)], idx_v)

        lanes = lax.iota(jnp.int32, 16)

        def gather_args(j, p):
            return (
                table_hbm.at[idx_v.at[pl.ds(j * CHUNK, CHUNK)]],
                rows_v.at[p],
                gsem.at[p],
            )

        pltpu.async_copy(*gather_args(0, 0))

        @pl.loop(0, N_CHUNKS, step=2)
        def _chunks(j0):
            for p in range(2):
                j = j0 + p
                b0 = b_base + j * CHUNK_B
                pltpu.make_async_copy(*gather_args(j, p)).wait()

                @pl.when(j + 1 < N_CHUNKS)
                def _next():
                    pltpu.async_copy(*gather_args(j + 1, (p + 1) % 2))

                @pl.when(j >= 2)
                def _drains():
                    @pl.loop(0, L)
                    def _drain(l):
                        pltpu.make_async_copy(
                            t_v.at[p, l],
                            out_hbm.at[
                                l, :, pl.ds(b_base + (j - 2) * CHUNK_B, CHUNK_B)
                            ],
                            wsem.at[p],
                        ).wait()

                @pl.loop(0, L)
                def _transpose(l):
                    for g in range(NGB):
                        rvec = (lanes + g * 16) * L + l
                        for c in range(EMB):
                            x = plsc.load_gather(
                                rows_v.at[p], [rvec, jnp.full((16,), c, jnp.int32)]
                            )
                            t_v[p, l, c, pl.ds(g * 16, 16)] = x

                @pl.loop(0, L)
                def _writeback(l):
                    pltpu.async_copy(
                        t_v.at[p, l],
                        out_hbm.at[l, :, pl.ds(b0, CHUNK_B)],
                        wsem.at[p],
                    )

        for j in range(N_CHUNKS - 2, N_CHUNKS):
            p = j % 2

            @pl.loop(0, L)
            def _drain_tail(l):
                pltpu.make_async_copy(
                    t_v.at[p, l],
                    out_hbm.at[l, :, pl.ds(b_base + j * CHUNK_B, CHUNK_B)],
                    wsem.at[p],
                ).wait()

    return gather_kernel


_gather = _make_gather()


@jax.jit
def kernel(indices, table):
    flat_idx = indices.reshape(N)
    out_lcb = _gather(flat_idx, table)
    return out_lcb.transpose(2, 0, 1)


# speedup vs baseline: 1.1439x; 1.1439x over previous
import functools

import jax
import jax.numpy as jnp
from jax import lax
from jax.experimental import pallas as pl
from jax.experimental.pallas import tpu as pltpu
from jax.experimental.pallas import tpu_sc as plsc

VOCAB = 1000000
EMB = 32
B = 16384
L = 20
N = B * L

NUM_CORES = 2
NUM_SUBCORES = 16
NW = NUM_CORES * NUM_SUBCORES
B_PER_W = B // NW
ROWS_PER_W = B_PER_W * L
CHUNK_B = 32
CHUNK = CHUNK_B * L
N_CHUNKS = B_PER_W // CHUNK_B
NGB = CHUNK_B // 16


def _make_gather():
    mesh = plsc.VectorSubcoreMesh(core_axis_name="c", subcore_axis_name="s")

    @functools.partial(
        pl.kernel,
        mesh=mesh,
        out_type=jax.ShapeDtypeStruct((L, EMB, B), jnp.float32),
        scratch_types=[
            pltpu.VMEM((ROWS_PER_W,), jnp.int32),
            pltpu.VMEM((2, CHUNK, EMB), jnp.float32),
            pltpu.VMEM((2, L, EMB, CHUNK_B), jnp.float32),
            pltpu.SemaphoreType.DMA((2,)),
            pltpu.SemaphoreType.DMA((2,)),
        ],
        compiler_params=pltpu.CompilerParams(
            use_tc_tiling_on_sc=False, needs_layout_passes=False
        ),
    )
    def gather_kernel(idx_hbm, table_hbm, out_hbm, idx_v, rows_v, t_v, gsem, wsem):
        wid = lax.axis_index("s") * NUM_CORES + lax.axis_index("c")
        base = wid * ROWS_PER_W
        b_base = wid * B_PER_W
        pltpu.sync_copy(idx_hbm.at[pl.ds(base, ROWS_PER_W)], idx_v)

        lanes = lax.iota(jnp.int32, 16)

        def gather_args(j, p):
            return (
                table_hbm.at[idx_v.at[pl.ds(j * CHUNK, CHUNK)]],
                rows_v.at[p],
                gsem.at[p],
            )

        pltpu.async_copy(*gather_args(0, 0))

        @pl.loop(0, N_CHUNKS, step=2)
        def _chunks(j0):
            for p in range(2):
                j = j0 + p
                b0 = b_base + j * CHUNK_B
                pltpu.make_async_copy(*gather_args(j, p)).wait()

                @pl.when(j + 1 < N_CHUNKS)
                def _next():
                    pltpu.async_copy(*gather_args(j + 1, (p + 1) % 2))

                @pl.when(j >= 2)
                def _drains():
                    @pl.loop(0, L)
                    def _drain(l):
                        pltpu.make_async_copy(
                            t_v.at[p, l],
                            out_hbm.at[
                                l, :, pl.ds(b_base + (j - 2) * CHUNK_B, CHUNK_B)
                            ],
                            wsem.at[p],
                        ).wait()

                @pl.loop(0, L)
                def _transpose(l):
                    for g in range(NGB):
                        rvec = (lanes + g * 16) * L + l
                        for cb in range(0, EMB, 8):
                            xs = [
                                plsc.load_gather(
                                    rows_v.at[p],
                                    [rvec, jnp.full((16,), cb + i, jnp.int32)],
                                )
                                for i in range(8)
                            ]
                            for i in range(8):
                                t_v[p, l, cb + i, pl.ds(g * 16, 16)] = xs[i]

                @pl.loop(0, L)
                def _writeback(l):
                    pltpu.async_copy(
                        t_v.at[p, l],
                        out_hbm.at[l, :, pl.ds(b0, CHUNK_B)],
                        wsem.at[p],
                    )

        for j in range(N_CHUNKS - 2, N_CHUNKS):
            p = j % 2

            @pl.loop(0, L)
            def _drain_tail(l):
                pltpu.make_async_copy(
                    t_v.at[p, l],
                    out_hbm.at[l, :, pl.ds(b_base + j * CHUNK_B, CHUNK_B)],
                    wsem.at[p],
                ).wait()

    return gather_kernel


_gather = _make_gather()


@jax.jit
def kernel(indices, table):
    flat_idx = indices.reshape(N)
    out_lcb = _gather(flat_idx, table)
    return out_lcb.transpose(2, 0, 1)
